# Initial kernel scaffold; baseline (speedup 1.0000x reference)
import jax
import jax.numpy as jnp
from jax.experimental import pallas as pl


def kernel(x, edge_index, edge_attr, W1, as1, ad1, We1, ae1, b1, g1, be1, W2, as2, ad2, We2, ae2, b2, g2, be2):
    raise NotImplementedError("write your pallas kernel here")


# speedup vs baseline: 1.2947x; 1.2947x over previous
import functools

import jax
import jax.numpy as jnp
from jax.experimental import pallas as pl

N = 50000
E = 800000
H1 = 4
C1 = 64
C2 = 64


def _ln_relu_kernel(x_ref, g_ref, b_ref, o_ref, *, relu):
    x = x_ref[...]
    mu = jnp.mean(x, axis=-1, keepdims=True)
    xc = x - mu
    var = jnp.mean(xc * xc, axis=-1, keepdims=True)
    y = xc * jax.lax.rsqrt(var + 1e-5) * g_ref[...] + b_ref[...]
    if relu:
        y = jnp.maximum(y, 0.0)
    o_ref[...] = y


def _ln(x, g, b, relu):
    n, d = x.shape
    blk = 1000
    return pl.pallas_call(
        functools.partial(_ln_relu_kernel, relu=relu),
        grid=(n // blk,),
        in_specs=[
            pl.BlockSpec((blk, d), lambda i: (i, 0)),
            pl.BlockSpec((1, d), lambda i: (0, 0)),
            pl.BlockSpec((1, d), lambda i: (0, 0)),
        ],
        out_specs=pl.BlockSpec((blk, d), lambda i: (i, 0)),
        out_shape=jax.ShapeDtypeStruct((n, d), x.dtype),
    )(x, g.reshape(1, d), b.reshape(1, d))


def _collapse(W, att, heads, C):
    return (W.reshape(-1, heads, C) * att[None]).sum(-1)


def _gat(x, src, dst, edge_attr, mean_ea, W, att_s, att_d, We, att_e, b, heads, C, concat):
    xh = (x @ W).reshape(N, heads, C)
    a_src = x @ _collapse(W, att_s, heads, C)
    a_dst = x @ _collapse(W, att_d, heads, C)
    Me = _collapse(We, att_e, heads, C)
    a_e = edge_attr @ Me
    a_e_loop = mean_ea @ Me

    al = a_src[src] + a_dst[dst] + a_e
    ex = jnp.exp(jax.nn.leaky_relu(al, 0.2))
    al_loop = a_src + a_dst + a_e_loop
    ex_loop = jnp.exp(jax.nn.leaky_relu(al_loop, 0.2))

    den = jax.ops.segment_sum(ex, dst, num_segments=N) + ex_loop + 1e-16
    msg = xh[src] * ex[:, :, None]
    outr = jax.ops.segment_sum(msg, dst, num_segments=N) + xh * ex_loop[:, :, None]
    out = outr / den[:, :, None]
    if concat:
        out = out.reshape(N, heads * C)
    else:
        out = out.mean(axis=1)
    return out + b


def kernel(x, edge_index, edge_attr, W1, as1, ad1, We1, ae1, b1, g1, be1, W2, as2, ad2, We2, ae2, b2, g2, be2):
    src, dst = edge_index[0], edge_index[1]
    ea_sum = jax.ops.segment_sum(edge_attr, dst, num_segments=N)
    cnt = jax.ops.segment_sum(jnp.ones((E,), jnp.float32), dst, num_segments=N)
    mean_ea = ea_sum / jnp.clip(cnt, 1.0, None)[:, None]

    h = _gat(x, src, dst, edge_attr, mean_ea, W1, as1, ad1, We1, ae1, b1, H1, C1, True)
    h = _ln(h, g1, be1, relu=True)
    h = _gat(h, src, dst, edge_attr, mean_ea, W2, as2, ad2, We2, ae2, b2, 1, C2, False)
    h = _ln(h, g2, be2, relu=False)
    return h


# speedup vs baseline: 14.1005x; 10.8908x over previous
import functools

import jax
import jax.numpy as jnp
from jax import lax
from jax.experimental import pallas as pl
from jax.experimental.pallas import tpu as pltpu
from jax.experimental.pallas import tpu_sc as plsc

N = 50000
E = 800000
H1 = 4
C1 = 64
C2 = 64

NC = 2
NS = 16
NW = NC * NS
LANES = 16
FB = 32

NPAD = 50048
RPT = NPAD // NS
ZR = 136
G = 512
SB = 128
NGPT = 49
EP = G * NW * NGPT
EPS = EP + G
NSB = G // SB


def _worker_groups(wid, do_group):
    @pl.loop(0, NGPT)
    def _(i):
        do_group(wid * NGPT + i, NSB)


def _make_sc_att(heads, with_ea):
    mesh = plsc.VectorSubcoreMesh(core_axis_name="c", subcore_axis_name="s")

    def body(*refs):
        (ts_hbm, td_hbm, src_hbm, dst_hbm, aep_hbm) = refs[:5]
        k = 5
        if with_ea:
            eap_hbm = refs[k]
            k += 1
        expad_hbm, dpart_hbm = refs[k], refs[k + 1]
        k += 2
        idx_s, idx_d, gs, gd, aev = refs[k:k + 5]
        k += 5
        if with_ea:
            eav = refs[k]
            k += 1
        zbuf, den = refs[k], refs[k + 1]

        c = lax.axis_index("c")
        s = lax.axis_index("s")
        wid = s * NC + c

        zeros = jnp.zeros((LANES,), jnp.float32)
        lane = lax.iota(jnp.int32, LANES)
        mask_ex = jnp.where(lane < heads, 1.0, 0.0).astype(jnp.float32)

        @pl.loop(0, ZR)
        def _(i):
            zbuf[i, pl.ds(0, LANES)] = zeros

        @pl.loop(0, RPT, step=ZR)
        def _(r0):
            pltpu.sync_copy(zbuf, den.at[pl.ds(s * RPT + r0, ZR)])

        plsc.subcore_barrier()

        def do_group(g, nsb):
            e0 = g * G
            r0 = g * (G // SB)
            ne = nsb * SB
            pltpu.sync_copy(src_hbm.at[pl.ds(r0, nsb)], idx_s.at[pl.ds(0, nsb)])
            pltpu.sync_copy(dst_hbm.at[pl.ds(r0, nsb)], idx_d.at[pl.ds(0, nsb)])
            pltpu.sync_copy(aep_hbm.at[pl.ds(e0, ne)], aev.at[pl.ds(0, ne)])
            if with_ea:
                pltpu.sync_copy(eap_hbm.at[pl.ds(e0, ne)], eav.at[pl.ds(0, ne)])
            for j in range(nsb):
                pltpu.sync_copy(ts_hbm.at[idx_s.at[j]], gs.at[pl.ds(j * SB, SB)])
                pltpu.sync_copy(td_hbm.at[idx_d.at[j]], gd.at[pl.ds(j * SB, SB)])

            @pl.loop(0, ne)
            def _(e):
                al = (gs[e, pl.ds(0, LANES)] + gd[e, pl.ds(0, LANES)]
                      + aev[e, pl.ds(0, LANES)])
                al = jnp.maximum(al, al * 0.2)
                exr = jnp.exp(al)
                gs[e, pl.ds(0, LANES)] = exr
                v = exr * mask_ex
                if with_ea:
                    eav[e, pl.ds(0, LANES)] = v + eav[e, pl.ds(0, LANES)]
                else:
                    gd[e, pl.ds(0, LANES)] = v

            pltpu.sync_copy(gs.at[pl.ds(0, ne)], expad_hbm.at[pl.ds(e0, ne)])
            vsrc = eav if with_ea else gd
            for j in range(nsb):
                pltpu.sync_copy(vsrc.at[pl.ds(j * SB, SB)],
                                den.at[idx_d.at[j]], add=True)

        _worker_groups(wid, do_group)

        plsc.subcore_barrier()
        pltpu.sync_copy(den.at[pl.ds(s * RPT, RPT)],
                        dpart_hbm.at[c, pl.ds(s * RPT, RPT)])

    scratch = [
        pltpu.VMEM((G // SB, SB), jnp.int32),
        pltpu.VMEM((G // SB, SB), jnp.int32),
        pltpu.VMEM((G, LANES), jnp.float32),
        pltpu.VMEM((G, LANES), jnp.float32),
        pltpu.VMEM((G, LANES), jnp.float32),
    ]
    if with_ea:
        scratch.append(pltpu.VMEM((G, LANES), jnp.float32))
    scratch += [
        pltpu.VMEM((ZR, LANES), jnp.float32),
        pltpu.VMEM_SHARED((NPAD, LANES), jnp.float32),
    ]
    return pl.kernel(
        body,
        out_type=(jax.ShapeDtypeStruct((EP, LANES), jnp.float32),
                  jax.ShapeDtypeStruct((NC, NPAD, LANES), jnp.float32)),
        mesh=mesh,
        compiler_params=pltpu.CompilerParams(use_tc_tiling_on_sc=False),
        scratch_types=scratch,
    )


_sc_att_l1 = _make_sc_att(H1, True)
_sc_att_l2 = _make_sc_att(1, False)


def _make_sc_agg(nblk):
    mesh = plsc.VectorSubcoreMesh(core_axis_name="c", subcore_axis_name="s")

    def body(xhb_hbm, src_hbm, dst_hbm, ext_hbm, out_hbm,
             idx_s, idx_d, exv, rows, zbuf, acc):
        c = lax.axis_index("c")
        s = lax.axis_index("s")
        wid = s * NC + c

        zeros = jnp.zeros((LANES,), jnp.float32)

        @pl.loop(0, ZR)
        def _(i):
            zbuf[i, pl.ds(0, LANES)] = zeros
            zbuf[i, pl.ds(LANES, LANES)] = zeros

        @pl.loop(0, nblk)
        def _(b):
            hb = b // (C1 // FB)

            @pl.loop(0, RPT, step=ZR)
            def _(r0):
                pltpu.sync_copy(zbuf, acc.at[pl.ds(s * RPT + r0, ZR)])

            plsc.subcore_barrier()

            def do_group(g, nsb):
                e0 = g * G
                r0 = g * (G // SB)
                pltpu.sync_copy(src_hbm.at[pl.ds(r0, nsb)], idx_s.at[pl.ds(0, nsb)])
                pltpu.sync_copy(dst_hbm.at[pl.ds(r0, nsb)], idx_d.at[pl.ds(0, nsb)])
                pltpu.sync_copy(ext_hbm.at[hb, pl.ds(e0, nsb * SB)],
                                exv.at[pl.ds(0, nsb * SB)])
                for j in range(nsb):
                    pltpu.sync_copy(xhb_hbm.at[b].at[idx_s.at[j]],
                                    rows.at[pl.ds(j * SB, SB)])

                @pl.loop(0, nsb * SB, step=LANES)
                def _(q):
                    ex16 = exv[pl.ds(q, LANES)]
                    for j in range(LANES):
                        w = ex16.at[jnp.full((LANES,), j, jnp.int32)].get(
                            mode="promise_in_bounds")
                        rows[q + j, pl.ds(0, LANES)] = (
                            rows[q + j, pl.ds(0, LANES)] * w)
                        rows[q + j, pl.ds(LANES, LANES)] = (
                            rows[q + j, pl.ds(LANES, LANES)] * w)

                for j in range(nsb):
                    pltpu.sync_copy(rows.at[pl.ds(j * SB, SB)],
                                    acc.at[idx_d.at[j]], add=True)

            _worker_groups(wid, do_group)

            plsc.subcore_barrier()

            pltpu.sync_copy(acc.at[pl.ds(s * RPT, RPT)],
                            out_hbm.at[c, b, pl.ds(s * RPT, RPT)])

            plsc.subcore_barrier()

    return pl.kernel(
        body,
        out_type=jax.ShapeDtypeStruct((NC, nblk, NPAD, FB), jnp.float32),
        mesh=mesh,
        compiler_params=pltpu.CompilerParams(use_tc_tiling_on_sc=False),
        scratch_types=[
            pltpu.VMEM((G // SB, SB), jnp.int32),
            pltpu.VMEM((G // SB, SB), jnp.int32),
            pltpu.VMEM((G,), jnp.float32),
            pltpu.VMEM((G, FB), jnp.float32),
            pltpu.VMEM((ZR, FB), jnp.float32),
            pltpu.VMEM_SHARED((NPAD, FB), jnp.float32),
        ],
    )


_sc_agg_l1 = _make_sc_agg(8)
_sc_agg_l2 = _make_sc_agg(2)


def _ln_relu_kernel(x_ref, g_ref, b_ref, o_ref, *, relu):
    x = x_ref[...]
    mu = jnp.mean(x, axis=-1, keepdims=True)
    xc = x - mu
    var = jnp.mean(xc * xc, axis=-1, keepdims=True)
    y = xc * jax.lax.rsqrt(var + 1e-5) * g_ref[...] + b_ref[...]
    if relu:
        y = jnp.maximum(y, 0.0)
    o_ref[...] = y


def _ln(x, g, b, relu):
    n, d = x.shape
    blk = 1000
    return pl.pallas_call(
        functools.partial(_ln_relu_kernel, relu=relu),
        grid=(n // blk,),
        in_specs=[
            pl.BlockSpec((blk, d), lambda i: (i, 0)),
            pl.BlockSpec((1, d), lambda i: (0, 0)),
            pl.BlockSpec((1, d), lambda i: (0, 0)),
        ],
        out_specs=pl.BlockSpec((blk, d), lambda i: (i, 0)),
        out_shape=jax.ShapeDtypeStruct((n, d), x.dtype),
    )(x, g.reshape(1, d), b.reshape(1, d))


def _combine_ln_kernel(outr_ref, xh_ref, exl_ref, den_ref, bias_ref,
                       g_ref, be_ref, o_ref, *, relu):
    y = (outr_ref[...] + xh_ref[...] * exl_ref[...]) / den_ref[...]
    x = y + bias_ref[...]
    mu = jnp.mean(x, axis=-1, keepdims=True)
    xc = x - mu
    var = jnp.mean(xc * xc, axis=-1, keepdims=True)
    y = xc * jax.lax.rsqrt(var + 1e-5) * g_ref[...] + be_ref[...]
    if relu:
        y = jnp.maximum(y, 0.0)
    o_ref[...] = y


def _combine_ln(outr, xh, exl_r, den_r, bias, g, be, relu):
    n, d = outr.shape
    blk = 1000
    row = lambda a: a.reshape(1, d)
    return pl.pallas_call(
        functools.partial(_combine_ln_kernel, relu=relu),
        grid=(n // blk,),
        in_specs=[pl.BlockSpec((blk, d), lambda i: (i, 0))] * 4
        + [pl.BlockSpec((1, d), lambda i: (0, 0))] * 3,
        out_specs=pl.BlockSpec((blk, d), lambda i: (i, 0)),
        out_shape=jax.ShapeDtypeStruct((n, d), jnp.float32),
    )(outr, xh, exl_r, den_r, row(bias), row(g), row(be))


def _mm_kernel(x_ref, w_ref, o_ref):
    o_ref[...] = jnp.dot(x_ref[...], w_ref[...],
                         preferred_element_type=jnp.float32)


def _mm(x, w, blk):
    n, kdim = x.shape
    m = w.shape[1]
    return pl.pallas_call(
        _mm_kernel,
        grid=(n // blk,),
        in_specs=[
            pl.BlockSpec((blk, kdim), lambda i: (i, 0)),
            pl.BlockSpec((kdim, m), lambda i: (0, 0)),
        ],
        out_specs=pl.BlockSpec((blk, m), lambda i: (i, 0)),
        out_shape=jax.ShapeDtypeStruct((n, m), jnp.float32),
    )(x, w)


def _collapse(W, att, heads, C):
    return (W.reshape(-1, heads, C) * att[None]).sum(-1)


def _pad16(a):
    return jnp.pad(a, ((0, 0), (0, LANES - a.shape[1])))


def _epad(a):
    return jnp.pad(a, ((0, EPS - a.shape[0]),) + ((0, 0),) * (a.ndim - 1))


def _gat(x, src2, dst2, edge_attr, mean_ea, W, att_s, att_d, We, att_e, b,
         heads, C, concat, sc_att, sc_agg, nblk):
    Wcat = jnp.concatenate(
        [W, _collapse(W, att_s, heads, C), _collapse(W, att_d, heads, C)],
        axis=1)
    xcat = _mm(x, Wcat, 2000)
    xh = xcat[:, :heads * C]
    a_src = xcat[:, heads * C:heads * C + heads]
    a_dst = xcat[:, heads * C + heads:]
    Me = _collapse(We, att_e, heads, C)
    aePad = _epad(_pad16(_mm(edge_attr, Me, 8000)))

    with_ea = mean_ea is None
    if with_ea:
        eaPad = _epad(jnp.concatenate(
            [jnp.zeros((E, heads), jnp.float32), edge_attr,
             jnp.ones((E, 1), jnp.float32),
             jnp.zeros((E, LANES - heads - 7), jnp.float32)], axis=1))
        exPad, dpart = sc_att(_pad16(a_src), _pad16(a_dst), src2, dst2,
                              aePad, eaPad)
        easum = dpart[:, :N, heads:heads + 6].sum(0)
        cnt = dpart[:, :N, heads + 6].sum(0)
        mean_ea = easum / jnp.clip(cnt, 1.0, None)[:, None]
    else:
        exPad, dpart = sc_att(_pad16(a_src), _pad16(a_dst), src2, dst2, aePad)

    ex = exPad[:E, :heads]
    den_e = dpart[:, :N, :heads].sum(0)

    a_e_loop = _mm(mean_ea, Me, 2000)
    al_loop = a_src + a_dst + a_e_loop
    ex_loop = jnp.exp(jax.nn.leaky_relu(al_loop, 0.2))
    den = den_e + ex_loop + 1e-16

    ext = jnp.pad(ex.T, ((0, 0), (0, EPS - E)))
    outp = sc_agg(xh.reshape(N, nblk, FB).transpose(1, 0, 2), src2, dst2, ext)
    outr = outp[:, :, :N].sum(axis=0).transpose(1, 0, 2).reshape(N, heads * C)
    return (outr, xh, jnp.repeat(ex_loop, C, axis=1),
            jnp.repeat(den, C, axis=1)), mean_ea


def kernel(x, edge_index, edge_attr, W1, as1, ad1, We1, ae1, b1, g1, be1,
           W2, as2, ad2, We2, ae2, b2, g2, be2):
    src2 = jnp.pad(edge_index[0], (0, EPS - E)).reshape(EPS // SB, SB)
    dst2 = jnp.pad(edge_index[1], (0, EPS - E),
                   constant_values=N).reshape(EPS // SB, SB)

    parts, mean_ea = _gat(x, src2, dst2, edge_attr, None, W1, as1, ad1, We1,
                          ae1, b1, H1, C1, True, _sc_att_l1, _sc_agg_l1, 8)
    h = _combine_ln(*parts, b1, g1, be1, relu=True)
    parts, _ = _gat(h, src2, dst2, edge_attr, mean_ea, W2, as2, ad2, We2,
                    ae2, b2, 1, C2, False, _sc_att_l2, _sc_agg_l2, 2)
    h = _combine_ln(*parts, b2, g2, be2, relu=False)
    return h


# speedup vs baseline: 14.1551x; 1.0039x over previous
import functools

import jax
import jax.numpy as jnp
from jax import lax
from jax.experimental import pallas as pl
from jax.experimental.pallas import tpu as pltpu
from jax.experimental.pallas import tpu_sc as plsc

N = 50000
E = 800000
H1 = 4
C1 = 64
C2 = 64

NC = 2
NS = 16
NW = NC * NS
LANES = 16
FB = 32

NPAD = 50048
RPT = NPAD // NS
ZR = 136
G = 512
SB = 128
NGPT = 49
EP = G * NW * NGPT
EPS = EP + G
NSB = G // SB


def _worker_groups(wid, do_group):
    @pl.loop(0, NGPT)
    def _(i):
        do_group(wid * NGPT + i, NSB)


def _make_sc_att(heads, with_ea):
    mesh = plsc.VectorSubcoreMesh(core_axis_name="c", subcore_axis_name="s")

    def body(*refs):
        (ts_hbm, td_hbm, sd_hbm, ae_hbm) = refs[:4]
        expad_hbm, dpart_hbm = refs[4], refs[5]
        k = 6
        idx, gs, gd, aev = refs[k:k + 4]
        k += 4
        zbuf, den = refs[k], refs[k + 1]

        c = lax.axis_index("c")
        s = lax.axis_index("s")
        wid = s * NC + c

        zeros = jnp.zeros((LANES,), jnp.float32)
        lane = lax.iota(jnp.int32, LANES)
        mask_ex = jnp.where(lane < heads, 1.0, 0.0).astype(jnp.float32)

        @pl.loop(0, ZR)
        def _(i):
            zbuf[i, pl.ds(0, LANES)] = zeros

        @pl.loop(0, RPT, step=ZR)
        def _(r0):
            pltpu.sync_copy(zbuf, den.at[pl.ds(s * RPT + r0, ZR)])

        plsc.subcore_barrier()

        nst = 2 if with_ea else 1

        def do_group(g, nsb):
            ne = nsb * SB
            pltpu.sync_copy(sd_hbm.at[pl.ds(g * 2 * NSB, 2 * nsb)],
                            idx.at[pl.ds(0, 2 * nsb)])
            pltpu.sync_copy(ae_hbm.at[pl.ds(g * nst * G, nst * ne)],
                            aev.at[pl.ds(0, nst * ne)])
            for j in range(nsb):
                pltpu.sync_copy(ts_hbm.at[idx.at[j]], gs.at[pl.ds(j * SB, SB)])
                pltpu.sync_copy(td_hbm.at[idx.at[nsb + j]],
                                gd.at[pl.ds(j * SB, SB)])

            @pl.loop(0, ne)
            def _(e):
                al = (gs[e, pl.ds(0, LANES)] + gd[e, pl.ds(0, LANES)]
                      + aev[e, pl.ds(0, LANES)])
                al = jnp.maximum(al, al * 0.2)
                exr = jnp.exp(al)
                gs[e, pl.ds(0, LANES)] = exr
                v = exr * mask_ex
                if with_ea:
                    aev[ne + e, pl.ds(0, LANES)] = (
                        v + aev[ne + e, pl.ds(0, LANES)])
                else:
                    gd[e, pl.ds(0, LANES)] = v

            pltpu.sync_copy(gs.at[pl.ds(0, ne)],
                            expad_hbm.at[pl.ds(g * G, ne)])
            voff = ne if with_ea else 0
            vsrc = aev if with_ea else gd
            for j in range(nsb):
                pltpu.sync_copy(vsrc.at[pl.ds(voff + j * SB, SB)],
                                den.at[idx.at[nsb + j]], add=True)

        _worker_groups(wid, do_group)

        plsc.subcore_barrier()
        pltpu.sync_copy(den.at[pl.ds(s * RPT, RPT)],
                        dpart_hbm.at[c, pl.ds(s * RPT, RPT)])

    scratch = [
        pltpu.VMEM((2 * G // SB, SB), jnp.int32),
        pltpu.VMEM((G, LANES), jnp.float32),
        pltpu.VMEM((G, LANES), jnp.float32),
        pltpu.VMEM(((2 if with_ea else 1) * G, LANES), jnp.float32),
        pltpu.VMEM((ZR, LANES), jnp.float32),
        pltpu.VMEM_SHARED((NPAD, LANES), jnp.float32),
    ]
    return pl.kernel(
        body,
        out_type=(jax.ShapeDtypeStruct((EP, LANES), jnp.float32),
                  jax.ShapeDtypeStruct((NC, NPAD, LANES), jnp.float32)),
        mesh=mesh,
        compiler_params=pltpu.CompilerParams(use_tc_tiling_on_sc=False),
        scratch_types=scratch,
    )


_sc_att_l1 = _make_sc_att(H1, True)
_sc_att_l2 = _make_sc_att(1, False)


def _make_sc_agg(nblk):
    mesh = plsc.VectorSubcoreMesh(core_axis_name="c", subcore_axis_name="s")

    def body(xhb_hbm, sdx_hbm, out_hbm, idx, rows, zbuf, acc):
        c = lax.axis_index("c")
        s = lax.axis_index("s")
        wid = s * NC + c

        zeros = jnp.zeros((LANES,), jnp.float32)

        @pl.loop(0, ZR)
        def _(i):
            zbuf[i, pl.ds(0, LANES)] = zeros
            zbuf[i, pl.ds(LANES, LANES)] = zeros

        @pl.loop(0, nblk)
        def _(b):
            hb = b // (C1 // FB)

            @pl.loop(0, RPT, step=ZR)
            def _(r0):
                pltpu.sync_copy(zbuf, acc.at[pl.ds(s * RPT + r0, ZR)])

            plsc.subcore_barrier()

            def do_group(g, nsb):
                pltpu.sync_copy(sdx_hbm.at[hb, pl.ds(g * 3 * NSB, 3 * nsb)],
                                idx.at[pl.ds(0, 3 * nsb)])
                for j in range(nsb):
                    pltpu.sync_copy(xhb_hbm.at[b].at[idx.at[j]],
                                    rows.at[pl.ds(j * SB, SB)])

                for j in range(nsb):
                    @pl.loop(j * SB, (j + 1) * SB, step=LANES)
                    def _(q):
                        ex16 = lax.bitcast_convert_type(
                            idx[2 * nsb + j, pl.ds(q - j * SB, LANES)],
                            jnp.float32)
                        for jj in range(LANES):
                            w = ex16.at[jnp.full((LANES,), jj, jnp.int32)].get(
                                mode="promise_in_bounds")
                            rows[q + jj, pl.ds(0, LANES)] = (
                                rows[q + jj, pl.ds(0, LANES)] * w)
                            rows[q + jj, pl.ds(LANES, LANES)] = (
                                rows[q + jj, pl.ds(LANES, LANES)] * w)

                for j in range(nsb):
                    pltpu.sync_copy(rows.at[pl.ds(j * SB, SB)],
                                    acc.at[idx.at[nsb + j]], add=True)

            _worker_groups(wid, do_group)

            plsc.subcore_barrier()

            pltpu.sync_copy(acc.at[pl.ds(s * RPT, RPT)],
                            out_hbm.at[c, b, pl.ds(s * RPT, RPT)])

            plsc.subcore_barrier()

    return pl.kernel(
        body,
        out_type=jax.ShapeDtypeStruct((NC, nblk, NPAD, FB), jnp.float32),
        mesh=mesh,
        compiler_params=pltpu.CompilerParams(use_tc_tiling_on_sc=False),
        scratch_types=[
            pltpu.VMEM((3 * G // SB, SB), jnp.int32),
            pltpu.VMEM((G, FB), jnp.float32),
            pltpu.VMEM((ZR, FB), jnp.float32),
            pltpu.VMEM_SHARED((NPAD, FB), jnp.float32),
        ],
    )


_sc_agg_l1 = _make_sc_agg(8)
_sc_agg_l2 = _make_sc_agg(2)


def _ln_relu_kernel(x_ref, g_ref, b_ref, o_ref, *, relu):
    x = x_ref[...]
    mu = jnp.mean(x, axis=-1, keepdims=True)
    xc = x - mu
    var = jnp.mean(xc * xc, axis=-1, keepdims=True)
    y = xc * jax.lax.rsqrt(var + 1e-5) * g_ref[...] + b_ref[...]
    if relu:
        y = jnp.maximum(y, 0.0)
    o_ref[...] = y


def _ln(x, g, b, relu):
    n, d = x.shape
    blk = 1000
    return pl.pallas_call(
        functools.partial(_ln_relu_kernel, relu=relu),
        grid=(n // blk,),
        in_specs=[
            pl.BlockSpec((blk, d), lambda i: (i, 0)),
            pl.BlockSpec((1, d), lambda i: (0, 0)),
            pl.BlockSpec((1, d), lambda i: (0, 0)),
        ],
        out_specs=pl.BlockSpec((blk, d), lambda i: (i, 0)),
        out_shape=jax.ShapeDtypeStruct((n, d), x.dtype),
    )(x, g.reshape(1, d), b.reshape(1, d))


def _combine_ln_kernel(outr_ref, xh_ref, exl_ref, den_ref, bias_ref,
                       g_ref, be_ref, o_ref, *, relu):
    y = (outr_ref[...] + xh_ref[...] * exl_ref[...]) / den_ref[...]
    x = y + bias_ref[...]
    mu = jnp.mean(x, axis=-1, keepdims=True)
    xc = x - mu
    var = jnp.mean(xc * xc, axis=-1, keepdims=True)
    y = xc * jax.lax.rsqrt(var + 1e-5) * g_ref[...] + be_ref[...]
    if relu:
        y = jnp.maximum(y, 0.0)
    o_ref[...] = y


def _combine_ln(outr, xh, exl_r, den_r, bias, g, be, relu):
    n, d = outr.shape
    blk = 1000
    row = lambda a: a.reshape(1, d)
    return pl.pallas_call(
        functools.partial(_combine_ln_kernel, relu=relu),
        grid=(n // blk,),
        in_specs=[pl.BlockSpec((blk, d), lambda i: (i, 0))] * 4
        + [pl.BlockSpec((1, d), lambda i: (0, 0))] * 3,
        out_specs=pl.BlockSpec((blk, d), lambda i: (i, 0)),
        out_shape=jax.ShapeDtypeStruct((n, d), jnp.float32),
    )(outr, xh, exl_r, den_r, row(bias), row(g), row(be))


def _mm_kernel(x_ref, w_ref, o_ref):
    o_ref[...] = jnp.dot(x_ref[...], w_ref[...],
                         preferred_element_type=jnp.float32)


def _mm(x, w, blk):
    n, kdim = x.shape
    m = w.shape[1]
    return pl.pallas_call(
        _mm_kernel,
        grid=(n // blk,),
        in_specs=[
            pl.BlockSpec((blk, kdim), lambda i: (i, 0)),
            pl.BlockSpec((kdim, m), lambda i: (0, 0)),
        ],
        out_specs=pl.BlockSpec((blk, m), lambda i: (i, 0)),
        out_shape=jax.ShapeDtypeStruct((n, m), jnp.float32),
    )(x, w)


def _collapse(W, att, heads, C):
    return (W.reshape(-1, heads, C) * att[None]).sum(-1)


def _pad16(a):
    return jnp.pad(a, ((0, 0), (0, LANES - a.shape[1])))


def _epad(a):
    return jnp.pad(a, ((0, EPS - a.shape[0]),) + ((0, 0),) * (a.ndim - 1))


def _gat(x, src2, srcp, dstp, edge_attr, mean_ea, W, att_s, att_d, We,
         att_e, b, heads, C, concat, sc_att, sc_agg, nblk):
    Wcat = jnp.concatenate(
        [W, _collapse(W, att_s, heads, C), _collapse(W, att_d, heads, C)],
        axis=1)
    xcat = _mm(x, Wcat, 2000)
    xh = xcat[:, :heads * C]
    a_src = xcat[:, heads * C:heads * C + heads]
    a_dst = xcat[:, heads * C + heads:]
    Me = _collapse(We, att_e, heads, C)
    aePad = _epad(_pad16(_mm(edge_attr, Me, 8000)))

    ngr = EPS // G
    with_ea = mean_ea is None
    if with_ea:
        eaPad = _epad(jnp.concatenate(
            [jnp.zeros((E, heads), jnp.float32), edge_attr,
             jnp.ones((E, 1), jnp.float32),
             jnp.zeros((E, LANES - heads - 7), jnp.float32)], axis=1))
        aepk = jnp.concatenate(
            [aePad.reshape(ngr, G, LANES), eaPad.reshape(ngr, G, LANES)],
            axis=1).reshape(ngr * 2 * G, LANES)
        exPad, dpart = sc_att(_pad16(a_src), _pad16(a_dst), src2, aepk)
        easum = dpart[:, :N, heads:heads + 6].sum(0)
        cnt = dpart[:, :N, heads + 6].sum(0)
        mean_ea = easum / jnp.clip(cnt, 1.0, None)[:, None]
    else:
        exPad, dpart = sc_att(_pad16(a_src), _pad16(a_dst), src2, aePad)

    ex = exPad[:E, :heads]
    den_e = dpart[:, :N, :heads].sum(0)

    a_e_loop = _mm(mean_ea, Me, 2000)
    al_loop = a_src + a_dst + a_e_loop
    ex_loop = jnp.exp(jax.nn.leaky_relu(al_loop, 0.2))
    den = den_e + ex_loop + 1e-16

    extp = jnp.pad(lax.bitcast_convert_type(ex, jnp.int32).T,
                   ((0, 0), (0, EPS - E))).reshape(heads, ngr, NSB, SB)
    sdx = jnp.concatenate(
        [jnp.broadcast_to(srcp[None], (heads, ngr, NSB, SB)),
         jnp.broadcast_to(dstp[None], (heads, ngr, NSB, SB)),
         extp], axis=2).reshape(heads, ngr * 3 * NSB, SB)
    outp = sc_agg(xh.reshape(N, nblk, FB).transpose(1, 0, 2), sdx)
    outr = outp[:, :, :N].sum(axis=0).transpose(1, 0, 2).reshape(N, heads * C)
    return (outr, xh, jnp.repeat(ex_loop, C, axis=1),
            jnp.repeat(den, C, axis=1)), mean_ea


def kernel(x, edge_index, edge_attr, W1, as1, ad1, We1, ae1, b1, g1, be1,
           W2, as2, ad2, We2, ae2, b2, g2, be2):
    ngr = EPS // G
    srcp = jnp.pad(edge_index[0], (0, EPS - E)).reshape(ngr, NSB, SB)
    dstp = jnp.pad(edge_index[1], (0, EPS - E),
                   constant_values=N).reshape(ngr, NSB, SB)
    src2 = jnp.concatenate([srcp, dstp], axis=1).reshape(ngr * 2 * NSB, SB)

    parts, mean_ea = _gat(x, src2, srcp, dstp, edge_attr, None, W1, as1,
                          ad1, We1, ae1, b1, H1, C1, True,
                          _sc_att_l1, _sc_agg_l1, 8)
    h = _combine_ln(*parts, b1, g1, be1, relu=True)
    parts, _ = _gat(h, src2, srcp, dstp, edge_attr, mean_ea, W2, as2, ad2,
                    We2, ae2, b2, 1, C2, False, _sc_att_l2, _sc_agg_l2, 2)
    h = _combine_ln(*parts, b2, g2, be2, relu=False)
    return h


# speedup vs baseline: 15.4219x; 1.0895x over previous
import functools

import jax
import jax.numpy as jnp
from jax import lax
from jax.experimental import pallas as pl
from jax.experimental.pallas import tpu as pltpu
from jax.experimental.pallas import tpu_sc as plsc

N = 50000
E = 800000
H1 = 4
C1 = 64
C2 = 64

NC = 2
NS = 16
NW = NC * NS
LANES = 16
FB = 32

NPAD = 50048
RPT = NPAD // NS
ZR = 136
G = 512
SB = 256
NGPT = 49
EP = G * NW * NGPT
EPS = EP + G
NSB = G // SB


def _worker_groups(wid, do_group):
    @pl.loop(0, NGPT)
    def _(i):
        do_group(wid * NGPT + i, NSB)


def _make_sc_att(heads, with_ea):
    mesh = plsc.VectorSubcoreMesh(core_axis_name="c", subcore_axis_name="s")

    def body(*refs):
        (ts_hbm, td_hbm, sd_hbm, ae_hbm) = refs[:4]
        expad_hbm, dpart_hbm = refs[4], refs[5]
        k = 6
        idx, gs, gd, aev = refs[k:k + 4]
        k += 4
        zbuf, den = refs[k], refs[k + 1]

        c = lax.axis_index("c")
        s = lax.axis_index("s")
        wid = s * NC + c

        zeros = jnp.zeros((LANES,), jnp.float32)
        lane = lax.iota(jnp.int32, LANES)
        mask_ex = jnp.where(lane < heads, 1.0, 0.0).astype(jnp.float32)

        @pl.loop(0, ZR)
        def _(i):
            zbuf[i, pl.ds(0, LANES)] = zeros

        @pl.loop(0, RPT, step=ZR)
        def _(r0):
            pltpu.sync_copy(zbuf, den.at[pl.ds(s * RPT + r0, ZR)])

        plsc.subcore_barrier()

        nst = 2 if with_ea else 1

        def do_group(g, nsb):
            ne = nsb * SB
            pltpu.sync_copy(sd_hbm.at[pl.ds(g * 2 * NSB, 2 * nsb)],
                            idx.at[pl.ds(0, 2 * nsb)])
            pltpu.sync_copy(ae_hbm.at[pl.ds(g * nst * G, nst * ne)],
                            aev.at[pl.ds(0, nst * ne)])
            for j in range(nsb):
                pltpu.sync_copy(ts_hbm.at[idx.at[j]], gs.at[pl.ds(j * SB, SB)])
                pltpu.sync_copy(td_hbm.at[idx.at[nsb + j]],
                                gd.at[pl.ds(j * SB, SB)])

            @pl.loop(0, ne, step=2)
            def _(e0):
                for u in range(2):
                    e = e0 + u
                    al = (gs[e, pl.ds(0, LANES)] + gd[e, pl.ds(0, LANES)]
                          + aev[e, pl.ds(0, LANES)])
                    al = jnp.maximum(al, al * 0.2)
                    exr = jnp.exp(al)
                    gs[e, pl.ds(0, LANES)] = exr
                    v = exr * mask_ex
                    if with_ea:
                        aev[ne + e, pl.ds(0, LANES)] = (
                            v + aev[ne + e, pl.ds(0, LANES)])
                    else:
                        gd[e, pl.ds(0, LANES)] = v

            pltpu.sync_copy(gs.at[pl.ds(0, ne)],
                            expad_hbm.at[pl.ds(g * G, ne)])
            voff = ne if with_ea else 0
            vsrc = aev if with_ea else gd
            for j in range(nsb):
                pltpu.sync_copy(vsrc.at[pl.ds(voff + j * SB, SB)],
                                den.at[idx.at[nsb + j]], add=True)

        _worker_groups(wid, do_group)

        plsc.subcore_barrier()
        pltpu.sync_copy(den.at[pl.ds(s * RPT, RPT)],
                        dpart_hbm.at[c, pl.ds(s * RPT, RPT)])

    scratch = [
        pltpu.VMEM((2 * G // SB, SB), jnp.int32),
        pltpu.VMEM((G, LANES), jnp.float32),
        pltpu.VMEM((G, LANES), jnp.float32),
        pltpu.VMEM(((2 if with_ea else 1) * G, LANES), jnp.float32),
        pltpu.VMEM((ZR, LANES), jnp.float32),
        pltpu.VMEM_SHARED((NPAD, LANES), jnp.float32),
    ]
    return pl.kernel(
        body,
        out_type=(jax.ShapeDtypeStruct((EP, LANES), jnp.float32),
                  jax.ShapeDtypeStruct((NC, NPAD, LANES), jnp.float32)),
        mesh=mesh,
        compiler_params=pltpu.CompilerParams(use_tc_tiling_on_sc=False),
        scratch_types=scratch,
    )


_sc_att_l1 = _make_sc_att(H1, True)
_sc_att_l2 = _make_sc_att(1, False)


def _make_sc_agg(nblk):
    mesh = plsc.VectorSubcoreMesh(core_axis_name="c", subcore_axis_name="s")

    def body(xhb_hbm, sdx_hbm, out_hbm, idx, rows, zbuf, acc):
        c = lax.axis_index("c")
        s = lax.axis_index("s")
        wid = s * NC + c

        zeros = jnp.zeros((LANES,), jnp.float32)

        @pl.loop(0, ZR)
        def _(i):
            zbuf[i, pl.ds(0, LANES)] = zeros
            zbuf[i, pl.ds(LANES, LANES)] = zeros

        @pl.loop(0, nblk)
        def _(b):
            hb = b // (C1 // FB)

            @pl.loop(0, RPT, step=ZR)
            def _(r0):
                pltpu.sync_copy(zbuf, acc.at[pl.ds(s * RPT + r0, ZR)])

            plsc.subcore_barrier()

            def do_group(g, nsb):
                pltpu.sync_copy(sdx_hbm.at[hb, pl.ds(g * 3 * NSB, 3 * nsb)],
                                idx.at[pl.ds(0, 3 * nsb)])
                for j in range(nsb):
                    pltpu.sync_copy(xhb_hbm.at[b].at[idx.at[j]],
                                    rows.at[pl.ds(j * SB, SB)])

                for j in range(nsb):
                    @pl.loop(j * SB, (j + 1) * SB, step=LANES)
                    def _(q):
                        ex16 = lax.bitcast_convert_type(
                            idx[2 * nsb + j, pl.ds(q - j * SB, LANES)],
                            jnp.float32)
                        for jj in range(LANES):
                            w = ex16.at[jnp.full((LANES,), jj, jnp.int32)].get(
                                mode="promise_in_bounds")
                            rows[q + jj, pl.ds(0, LANES)] = (
                                rows[q + jj, pl.ds(0, LANES)] * w)
                            rows[q + jj, pl.ds(LANES, LANES)] = (
                                rows[q + jj, pl.ds(LANES, LANES)] * w)

                for j in range(nsb):
                    pltpu.sync_copy(rows.at[pl.ds(j * SB, SB)],
                                    acc.at[idx.at[nsb + j]], add=True)

            _worker_groups(wid, do_group)

            plsc.subcore_barrier()

            pltpu.sync_copy(acc.at[pl.ds(s * RPT, RPT)],
                            out_hbm.at[c, b, pl.ds(s * RPT, RPT)])

            plsc.subcore_barrier()

    return pl.kernel(
        body,
        out_type=jax.ShapeDtypeStruct((NC, nblk, NPAD, FB), jnp.float32),
        mesh=mesh,
        compiler_params=pltpu.CompilerParams(use_tc_tiling_on_sc=False),
        scratch_types=[
            pltpu.VMEM((3 * G // SB, SB), jnp.int32),
            pltpu.VMEM((G, FB), jnp.float32),
            pltpu.VMEM((ZR, FB), jnp.float32),
            pltpu.VMEM_SHARED((NPAD, FB), jnp.float32),
        ],
    )


_sc_agg_l1 = _make_sc_agg(8)
_sc_agg_l2 = _make_sc_agg(2)


def _ln_relu_kernel(x_ref, g_ref, b_ref, o_ref, *, relu):
    x = x_ref[...]
    mu = jnp.mean(x, axis=-1, keepdims=True)
    xc = x - mu
    var = jnp.mean(xc * xc, axis=-1, keepdims=True)
    y = xc * jax.lax.rsqrt(var + 1e-5) * g_ref[...] + b_ref[...]
    if relu:
        y = jnp.maximum(y, 0.0)
    o_ref[...] = y


def _ln(x, g, b, relu):
    n, d = x.shape
    blk = 1000
    return pl.pallas_call(
        functools.partial(_ln_relu_kernel, relu=relu),
        grid=(n // blk,),
        in_specs=[
            pl.BlockSpec((blk, d), lambda i: (i, 0)),
            pl.BlockSpec((1, d), lambda i: (0, 0)),
            pl.BlockSpec((1, d), lambda i: (0, 0)),
        ],
        out_specs=pl.BlockSpec((blk, d), lambda i: (i, 0)),
        out_shape=jax.ShapeDtypeStruct((n, d), x.dtype),
    )(x, g.reshape(1, d), b.reshape(1, d))


def _combine_ln_kernel(outr_ref, xh_ref, exl_ref, den_ref, bias_ref,
                       g_ref, be_ref, o_ref, *, relu):
    y = (outr_ref[...] + xh_ref[...] * exl_ref[...]) / den_ref[...]
    x = y + bias_ref[...]
    mu = jnp.mean(x, axis=-1, keepdims=True)
    xc = x - mu
    var = jnp.mean(xc * xc, axis=-1, keepdims=True)
    y = xc * jax.lax.rsqrt(var + 1e-5) * g_ref[...] + be_ref[...]
    if relu:
        y = jnp.maximum(y, 0.0)
    o_ref[...] = y


def _combine_ln(outr, xh, exl_r, den_r, bias, g, be, relu):
    n, d = outr.shape
    blk = 1000
    row = lambda a: a.reshape(1, d)
    return pl.pallas_call(
        functools.partial(_combine_ln_kernel, relu=relu),
        grid=(n // blk,),
        in_specs=[pl.BlockSpec((blk, d), lambda i: (i, 0))] * 4
        + [pl.BlockSpec((1, d), lambda i: (0, 0))] * 3,
        out_specs=pl.BlockSpec((blk, d), lambda i: (i, 0)),
        out_shape=jax.ShapeDtypeStruct((n, d), jnp.float32),
    )(outr, xh, exl_r, den_r, row(bias), row(g), row(be))


def _mm_kernel(x_ref, w_ref, o_ref):
    o_ref[...] = jnp.dot(x_ref[...], w_ref[...],
                         preferred_element_type=jnp.float32)


def _mm(x, w, blk):
    n, kdim = x.shape
    m = w.shape[1]
    return pl.pallas_call(
        _mm_kernel,
        grid=(n // blk,),
        in_specs=[
            pl.BlockSpec((blk, kdim), lambda i: (i, 0)),
            pl.BlockSpec((kdim, m), lambda i: (0, 0)),
        ],
        out_specs=pl.BlockSpec((blk, m), lambda i: (i, 0)),
        out_shape=jax.ShapeDtypeStruct((n, m), jnp.float32),
    )(x, w)


def _collapse(W, att, heads, C):
    return (W.reshape(-1, heads, C) * att[None]).sum(-1)


def _pad16(a):
    return jnp.pad(a, ((0, 0), (0, LANES - a.shape[1])))


def _epad(a):
    return jnp.pad(a, ((0, EPS - a.shape[0]),) + ((0, 0),) * (a.ndim - 1))


def _gat(x, src2, srcp, dstp, edge_attr, mean_ea, W, att_s, att_d, We,
         att_e, b, heads, C, concat, sc_att, sc_agg, nblk):
    Wcat = jnp.concatenate(
        [W, _collapse(W, att_s, heads, C), _collapse(W, att_d, heads, C)],
        axis=1)
    xcat = _mm(x, Wcat, 2000)
    xh = xcat[:, :heads * C]
    a_src = xcat[:, heads * C:heads * C + heads]
    a_dst = xcat[:, heads * C + heads:]
    Me = _collapse(We, att_e, heads, C)
    aePad = _epad(_pad16(_mm(edge_attr, Me, 8000)))

    ngr = EPS // G
    with_ea = mean_ea is None
    if with_ea:
        eaPad = _epad(jnp.concatenate(
            [jnp.zeros((E, heads), jnp.float32), edge_attr,
             jnp.ones((E, 1), jnp.float32),
             jnp.zeros((E, LANES - heads - 7), jnp.float32)], axis=1))
        aepk = jnp.concatenate(
            [aePad.reshape(ngr, G, LANES), eaPad.reshape(ngr, G, LANES)],
            axis=1).reshape(ngr * 2 * G, LANES)
        exPad, dpart = sc_att(_pad16(a_src), _pad16(a_dst), src2, aepk)
        easum = dpart[:, :N, heads:heads + 6].sum(0)
        cnt = dpart[:, :N, heads + 6].sum(0)
        mean_ea = easum / jnp.clip(cnt, 1.0, None)[:, None]
    else:
        exPad, dpart = sc_att(_pad16(a_src), _pad16(a_dst), src2, aePad)

    ex = exPad[:E, :heads]
    den_e = dpart[:, :N, :heads].sum(0)

    a_e_loop = _mm(mean_ea, Me, 2000)
    al_loop = a_src + a_dst + a_e_loop
    ex_loop = jnp.exp(jax.nn.leaky_relu(al_loop, 0.2))
    den = den_e + ex_loop + 1e-16

    extp = jnp.pad(lax.bitcast_convert_type(ex, jnp.int32).T,
                   ((0, 0), (0, EPS - E))).reshape(heads, ngr, NSB, SB)
    sdx = jnp.concatenate(
        [jnp.broadcast_to(srcp[None], (heads, ngr, NSB, SB)),
         jnp.broadcast_to(dstp[None], (heads, ngr, NSB, SB)),
         extp], axis=2).reshape(heads, ngr * 3 * NSB, SB)
    outp = sc_agg(xh.reshape(N, nblk, FB).transpose(1, 0, 2), sdx)
    outr = outp[:, :, :N].sum(axis=0).transpose(1, 0, 2).reshape(N, heads * C)
    return (outr, xh, jnp.repeat(ex_loop, C, axis=1),
            jnp.repeat(den, C, axis=1)), mean_ea


def kernel(x, edge_index, edge_attr, W1, as1, ad1, We1, ae1, b1, g1, be1,
           W2, as2, ad2, We2, ae2, b2, g2, be2):
    ngr = EPS // G
    srcp = jnp.pad(edge_index[0], (0, EPS - E)).reshape(ngr, NSB, SB)
    dstp = jnp.pad(edge_index[1], (0, EPS - E),
                   constant_values=N).reshape(ngr, NSB, SB)
    src2 = jnp.concatenate([srcp, dstp], axis=1).reshape(ngr * 2 * NSB, SB)

    parts, mean_ea = _gat(x, src2, srcp, dstp, edge_attr, None, W1, as1,
                          ad1, We1, ae1, b1, H1, C1, True,
                          _sc_att_l1, _sc_agg_l1, 8)
    h = _combine_ln(*parts, b1, g1, be1, relu=True)
    parts, _ = _gat(h, src2, srcp, dstp, edge_attr, mean_ea, W2, as2, ad2,
                    We2, ae2, b2, 1, C2, False, _sc_att_l2, _sc_agg_l2, 2)
    h = _combine_ln(*parts, b2, g2, be2, relu=False)
    return h


# speedup vs baseline: 16.9189x; 1.0971x over previous
import functools

import jax
import jax.numpy as jnp
from jax import lax
from jax.experimental import pallas as pl
from jax.experimental.pallas import tpu as pltpu
from jax.experimental.pallas import tpu_sc as plsc

N = 50000
E = 800000
H1 = 4
C1 = 64
C2 = 64

NC = 2
NS = 16
NW = NC * NS
LANES = 16
FB = 32

NPAD = 50048
RPT = NPAD // NS
ZR = 136
G = 512
SB = 512
NGPT = 49
EP = G * NW * NGPT
EPS = EP + G
NSB = G // SB


def _worker_groups(wid, do_group):
    @pl.loop(0, NGPT)
    def _(i):
        do_group(wid * NGPT + i, NSB)


def _make_sc_att(heads, with_ea):
    mesh = plsc.VectorSubcoreMesh(core_axis_name="c", subcore_axis_name="s")

    def body(*refs):
        (ts_hbm, td_hbm, sd_hbm, ae_hbm) = refs[:4]
        expad_hbm, dpart_hbm = refs[4], refs[5]
        k = 6
        idx, gs, gd, aev = refs[k:k + 4]
        k += 4
        zbuf, den = refs[k], refs[k + 1]

        c = lax.axis_index("c")
        s = lax.axis_index("s")
        wid = s * NC + c

        zeros = jnp.zeros((LANES,), jnp.float32)
        lane = lax.iota(jnp.int32, LANES)
        mask_ex = jnp.where(lane < heads, 1.0, 0.0).astype(jnp.float32)

        @pl.loop(0, ZR)
        def _(i):
            zbuf[i, pl.ds(0, LANES)] = zeros

        @pl.loop(0, RPT, step=ZR)
        def _(r0):
            pltpu.sync_copy(zbuf, den.at[pl.ds(s * RPT + r0, ZR)])

        plsc.subcore_barrier()

        nst = 2 if with_ea else 1

        def do_group(g, nsb):
            ne = nsb * SB
            pltpu.sync_copy(sd_hbm.at[pl.ds(g * 2 * NSB, 2 * nsb)],
                            idx.at[pl.ds(0, 2 * nsb)])
            pltpu.sync_copy(ae_hbm.at[pl.ds(g * nst * G, nst * ne)],
                            aev.at[pl.ds(0, nst * ne)])
            for j in range(nsb):
                pltpu.sync_copy(ts_hbm.at[idx.at[j]], gs.at[pl.ds(j * SB, SB)])
                pltpu.sync_copy(td_hbm.at[idx.at[nsb + j]],
                                gd.at[pl.ds(j * SB, SB)])

            @pl.loop(0, ne, step=2)
            def _(e0):
                for u in range(2):
                    e = e0 + u
                    al = (gs[e, pl.ds(0, LANES)] + gd[e, pl.ds(0, LANES)]
                          + aev[e, pl.ds(0, LANES)])
                    al = jnp.maximum(al, al * 0.2)
                    exr = jnp.exp(al)
                    gs[e, pl.ds(0, LANES)] = exr
                    v = exr * mask_ex
                    if with_ea:
                        aev[ne + e, pl.ds(0, LANES)] = (
                            v + aev[ne + e, pl.ds(0, LANES)])
                    else:
                        gd[e, pl.ds(0, LANES)] = v

            pltpu.sync_copy(gs.at[pl.ds(0, ne)],
                            expad_hbm.at[pl.ds(g * G, ne)])
            voff = ne if with_ea else 0
            vsrc = aev if with_ea else gd
            for j in range(nsb):
                pltpu.sync_copy(vsrc.at[pl.ds(voff + j * SB, SB)],
                                den.at[idx.at[nsb + j]], add=True)

        _worker_groups(wid, do_group)

        plsc.subcore_barrier()
        pltpu.sync_copy(den.at[pl.ds(s * RPT, RPT)],
                        dpart_hbm.at[c, pl.ds(s * RPT, RPT)])

    scratch = [
        pltpu.VMEM((2 * G // SB, SB), jnp.int32),
        pltpu.VMEM((G, LANES), jnp.float32),
        pltpu.VMEM((G, LANES), jnp.float32),
        pltpu.VMEM(((2 if with_ea else 1) * G, LANES), jnp.float32),
        pltpu.VMEM((ZR, LANES), jnp.float32),
        pltpu.VMEM_SHARED((NPAD, LANES), jnp.float32),
    ]
    return pl.kernel(
        body,
        out_type=(jax.ShapeDtypeStruct((EP, LANES), jnp.float32),
                  jax.ShapeDtypeStruct((NC, NPAD, LANES), jnp.float32)),
        mesh=mesh,
        compiler_params=pltpu.CompilerParams(use_tc_tiling_on_sc=False),
        scratch_types=scratch,
    )


_sc_att_l1 = _make_sc_att(H1, True)
_sc_att_l2 = _make_sc_att(1, False)


def _make_sc_agg(nblk):
    mesh = plsc.VectorSubcoreMesh(core_axis_name="c", subcore_axis_name="s")

    def body(xhb_hbm, sdx_hbm, out_hbm, idx, rows, zbuf, acc):
        c = lax.axis_index("c")
        s = lax.axis_index("s")
        wid = s * NC + c

        zeros = jnp.zeros((LANES,), jnp.float32)

        @pl.loop(0, ZR)
        def _(i):
            zbuf[i, pl.ds(0, LANES)] = zeros
            zbuf[i, pl.ds(LANES, LANES)] = zeros

        @pl.loop(0, nblk)
        def _(b):
            hb = b // (C1 // FB)

            @pl.loop(0, RPT, step=ZR)
            def _(r0):
                pltpu.sync_copy(zbuf, acc.at[pl.ds(s * RPT + r0, ZR)])

            plsc.subcore_barrier()

            def do_group(g, nsb):
                pltpu.sync_copy(sdx_hbm.at[hb, pl.ds(g * 3 * NSB, 3 * nsb)],
                                idx.at[pl.ds(0, 3 * nsb)])
                for j in range(nsb):
                    pltpu.sync_copy(xhb_hbm.at[b].at[idx.at[j]],
                                    rows.at[pl.ds(j * SB, SB)])

                for j in range(nsb):
                    @pl.loop(j * SB, (j + 1) * SB, step=LANES)
                    def _(q):
                        ex16 = lax.bitcast_convert_type(
                            idx[2 * nsb + j, pl.ds(q - j * SB, LANES)],
                            jnp.float32)
                        for jj in range(LANES):
                            w = ex16.at[jnp.full((LANES,), jj, jnp.int32)].get(
                                mode="promise_in_bounds")
                            rows[q + jj, pl.ds(0, LANES)] = (
                                rows[q + jj, pl.ds(0, LANES)] * w)
                            rows[q + jj, pl.ds(LANES, LANES)] = (
                                rows[q + jj, pl.ds(LANES, LANES)] * w)

                for j in range(nsb):
                    pltpu.sync_copy(rows.at[pl.ds(j * SB, SB)],
                                    acc.at[idx.at[nsb + j]], add=True)

            _worker_groups(wid, do_group)

            plsc.subcore_barrier()

            pltpu.sync_copy(acc.at[pl.ds(s * RPT, RPT)],
                            out_hbm.at[c, b, pl.ds(s * RPT, RPT)])

            plsc.subcore_barrier()

    return pl.kernel(
        body,
        out_type=jax.ShapeDtypeStruct((NC, nblk, NPAD, FB), jnp.float32),
        mesh=mesh,
        compiler_params=pltpu.CompilerParams(use_tc_tiling_on_sc=False),
        scratch_types=[
            pltpu.VMEM((3 * G // SB, SB), jnp.int32),
            pltpu.VMEM((G, FB), jnp.float32),
            pltpu.VMEM((ZR, FB), jnp.float32),
            pltpu.VMEM_SHARED((NPAD, FB), jnp.float32),
        ],
    )


_sc_agg_l1 = _make_sc_agg(8)
_sc_agg_l2 = _make_sc_agg(2)


def _ln_relu_kernel(x_ref, g_ref, b_ref, o_ref, *, relu):
    x = x_ref[...]
    mu = jnp.mean(x, axis=-1, keepdims=True)
    xc = x - mu
    var = jnp.mean(xc * xc, axis=-1, keepdims=True)
    y = xc * jax.lax.rsqrt(var + 1e-5) * g_ref[...] + b_ref[...]
    if relu:
        y = jnp.maximum(y, 0.0)
    o_ref[...] = y


def _ln(x, g, b, relu):
    n, d = x.shape
    blk = 1000
    return pl.pallas_call(
        functools.partial(_ln_relu_kernel, relu=relu),
        grid=(n // blk,),
        in_specs=[
            pl.BlockSpec((blk, d), lambda i: (i, 0)),
            pl.BlockSpec((1, d), lambda i: (0, 0)),
            pl.BlockSpec((1, d), lambda i: (0, 0)),
        ],
        out_specs=pl.BlockSpec((blk, d), lambda i: (i, 0)),
        out_shape=jax.ShapeDtypeStruct((n, d), x.dtype),
    )(x, g.reshape(1, d), b.reshape(1, d))


def _combine_ln_kernel(outr_ref, xh_ref, exl_ref, den_ref, bias_ref,
                       g_ref, be_ref, o_ref, *, relu):
    y = (outr_ref[...] + xh_ref[...] * exl_ref[...]) / den_ref[...]
    x = y + bias_ref[...]
    mu = jnp.mean(x, axis=-1, keepdims=True)
    xc = x - mu
    var = jnp.mean(xc * xc, axis=-1, keepdims=True)
    y = xc * jax.lax.rsqrt(var + 1e-5) * g_ref[...] + be_ref[...]
    if relu:
        y = jnp.maximum(y, 0.0)
    o_ref[...] = y


def _combine_ln(outr, xh, exl_r, den_r, bias, g, be, relu):
    n, d = outr.shape
    blk = 1000
    row = lambda a: a.reshape(1, d)
    return pl.pallas_call(
        functools.partial(_combine_ln_kernel, relu=relu),
        grid=(n // blk,),
        in_specs=[pl.BlockSpec((blk, d), lambda i: (i, 0))] * 4
        + [pl.BlockSpec((1, d), lambda i: (0, 0))] * 3,
        out_specs=pl.BlockSpec((blk, d), lambda i: (i, 0)),
        out_shape=jax.ShapeDtypeStruct((n, d), jnp.float32),
    )(outr, xh, exl_r, den_r, row(bias), row(g), row(be))


def _mm_kernel(x_ref, w_ref, o_ref):
    o_ref[...] = jnp.dot(x_ref[...], w_ref[...],
                         preferred_element_type=jnp.float32)


def _mm(x, w, blk):
    n, kdim = x.shape
    m = w.shape[1]
    return pl.pallas_call(
        _mm_kernel,
        grid=(n // blk,),
        in_specs=[
            pl.BlockSpec((blk, kdim), lambda i: (i, 0)),
            pl.BlockSpec((kdim, m), lambda i: (0, 0)),
        ],
        out_specs=pl.BlockSpec((blk, m), lambda i: (i, 0)),
        out_shape=jax.ShapeDtypeStruct((n, m), jnp.float32),
    )(x, w)


def _collapse(W, att, heads, C):
    return (W.reshape(-1, heads, C) * att[None]).sum(-1)


def _pad16(a):
    return jnp.pad(a, ((0, 0), (0, LANES - a.shape[1])))


def _epad(a):
    return jnp.pad(a, ((0, EPS - a.shape[0]),) + ((0, 0),) * (a.ndim - 1))


def _gat(x, src2, srcp, dstp, edge_attr, mean_ea, W, att_s, att_d, We,
         att_e, b, heads, C, concat, sc_att, sc_agg, nblk):
    Wcat = jnp.concatenate(
        [W, _collapse(W, att_s, heads, C), _collapse(W, att_d, heads, C)],
        axis=1)
    xcat = _mm(x, Wcat, 2000)
    xh = xcat[:, :heads * C]
    a_src = xcat[:, heads * C:heads * C + heads]
    a_dst = xcat[:, heads * C + heads:]
    Me = _collapse(We, att_e, heads, C)
    aePad = _epad(_pad16(_mm(edge_attr, Me, 8000)))

    ngr = EPS // G
    with_ea = mean_ea is None
    if with_ea:
        eaPad = _epad(jnp.concatenate(
            [jnp.zeros((E, heads), jnp.float32), edge_attr,
             jnp.ones((E, 1), jnp.float32),
             jnp.zeros((E, LANES - heads - 7), jnp.float32)], axis=1))
        aepk = jnp.concatenate(
            [aePad.reshape(ngr, G, LANES), eaPad.reshape(ngr, G, LANES)],
            axis=1).reshape(ngr * 2 * G, LANES)
        exPad, dpart = sc_att(_pad16(a_src), _pad16(a_dst), src2, aepk)
        easum = dpart[:, :N, heads:heads + 6].sum(0)
        cnt = dpart[:, :N, heads + 6].sum(0)
        mean_ea = easum / jnp.clip(cnt, 1.0, None)[:, None]
    else:
        exPad, dpart = sc_att(_pad16(a_src), _pad16(a_dst), src2, aePad)

    ex = exPad[:E, :heads]
    den_e = dpart[:, :N, :heads].sum(0)

    a_e_loop = _mm(mean_ea, Me, 2000)
    al_loop = a_src + a_dst + a_e_loop
    ex_loop = jnp.exp(jax.nn.leaky_relu(al_loop, 0.2))
    den = den_e + ex_loop + 1e-16

    extp = jnp.pad(lax.bitcast_convert_type(ex, jnp.int32).T,
                   ((0, 0), (0, EPS - E))).reshape(heads, ngr, NSB, SB)
    sdx = jnp.concatenate(
        [jnp.broadcast_to(srcp[None], (heads, ngr, NSB, SB)),
         jnp.broadcast_to(dstp[None], (heads, ngr, NSB, SB)),
         extp], axis=2).reshape(heads, ngr * 3 * NSB, SB)
    outp = sc_agg(xh.reshape(N, nblk, FB).transpose(1, 0, 2), sdx)
    outr = outp[:, :, :N].sum(axis=0).transpose(1, 0, 2).reshape(N, heads * C)
    return (outr, xh, jnp.repeat(ex_loop, C, axis=1),
            jnp.repeat(den, C, axis=1)), mean_ea


def kernel(x, edge_index, edge_attr, W1, as1, ad1, We1, ae1, b1, g1, be1,
           W2, as2, ad2, We2, ae2, b2, g2, be2):
    ngr = EPS // G
    srcp = jnp.pad(edge_index[0], (0, EPS - E)).reshape(ngr, NSB, SB)
    dstp = jnp.pad(edge_index[1], (0, EPS - E),
                   constant_values=N).reshape(ngr, NSB, SB)
    src2 = jnp.concatenate([srcp, dstp], axis=1).reshape(ngr * 2 * NSB, SB)

    parts, mean_ea = _gat(x, src2, srcp, dstp, edge_attr, None, W1, as1,
                          ad1, We1, ae1, b1, H1, C1, True,
                          _sc_att_l1, _sc_agg_l1, 8)
    h = _combine_ln(*parts, b1, g1, be1, relu=True)
    parts, _ = _gat(h, src2, srcp, dstp, edge_attr, mean_ea, W2, as2, ad2,
                    We2, ae2, b2, 1, C2, False, _sc_att_l2, _sc_agg_l2, 2)
    h = _combine_ln(*parts, b2, g2, be2, relu=False)
    return h
